# baseline (speedup 1.0000x reference)
import functools

import jax
import jax.numpy as jnp
from jax import lax
from jax.experimental import pallas as pl
from jax.experimental.pallas import tpu as pltpu
from jax.experimental.pallas import tpu_sc as plsc

N_NODES = 10000
N_REL = 500
ENT_IN = 128
NHID = 64
NHEADS = 2
OUT_DIM = 128
ALPHA = 0.2
N_EDGES = 160000
N_NHOP = 20000
E_TOT = N_EDGES + N_NHOP

NPAD = 10112
RPAD = 512
PAD_NODE = N_NODES
PAD_TYPE = N_REL

C = 64
E_PAD = 180224
NSL1, NCH1 = 16, 176
ZR = NPAD // 16

@functools.cache
def _make_edge_kernel(F, nsl, nch):
    NB = F // 16
    NG = C // 16

    def body(e_hbm, s0_hbm, s1_hbm, sr_hbm, p1_hbm, pr_hbm,
             h1_out, rs_out,
             s0_t, s1_t, sr_t, rowsum_t, pr_t,
             eb0, eb1, eb2, eb3, sb0, sb1, dx0, dx1,
             rows0, rows1, cn0, cn1, e_b,
             h1_sh, esem0, esem1, esem2, esem3, gsem0, gsem1, ssem0, ssem1):
        cid = lax.axis_index("c")
        sid = lax.axis_index("s")
        base = sid * nch
        noff = cid * NPAD
        ebs = (eb0, eb1, eb2, eb3)
        esems = (esem0, esem1, esem2, esem3)
        sbs = (sb0, sb1)
        gsems = (gsem0, gsem1)
        rowss = (rows0, rows1)
        dxs = (dx0, dx1)
        cns = (cn0, cn1)
        ssems = (ssem0, ssem1)

        pltpu.sync_copy(s0_hbm.at[pl.ds(cid * NPAD, NPAD)], s0_t)
        pltpu.sync_copy(s1_hbm.at[pl.ds(cid * NPAD, NPAD)], s1_t)
        pltpu.sync_copy(sr_hbm.at[pl.ds(cid * RPAD, RPAD)], sr_t)
        pltpu.sync_copy(pr_hbm.at[cid], pr_t)

        zv = jnp.zeros((16,), jnp.float32)

        def zero_rowsum(i, carry):
            rowsum_t[pl.ds(i * 16, 16)] = zv
            return carry

        lax.fori_loop(0, NPAD // 16, zero_rowsum, 0)

        def zero_cn0(i, carry):
            r = i // NB
            b = i % NB
            cn0[r, pl.ds(b * 16, 16)] = zv
            return carry

        lax.fori_loop(0, 2 * C * NB, zero_cn0, 0)

        nfull = ZR // (2 * C)
        for k in range(nfull):
            pltpu.sync_copy(cn0, h1_sh.at[pl.ds(sid * ZR + k * 2 * C, 2 * C)])
        rem = ZR - nfull * 2 * C
        if rem:
            pltpu.sync_copy(cn0.at[pl.ds(0, rem)],
                            h1_sh.at[pl.ds(sid * ZR + nfull * 2 * C, rem)])
        plsc.subcore_barrier()

        padv = jnp.full((16,), PAD_NODE, jnp.int32)
        for k in range(2 * NG):
            dx0[pl.ds(k * 16, 16)] = padv
            dx1[pl.ds(k * 16, 16)] = padv
        pltpu.async_copy(cn0, h1_sh.at[dx0], ssem0, add=True)
        pltpu.async_copy(cn1, h1_sh.at[dx1], ssem1, add=True)
        for i in range(4):
            pltpu.async_copy(e_hbm.at[base + i], ebs[i], esems[i])
        pltpu.make_async_copy(e_hbm.at[base], eb0, esem0).wait()
        for k in range(NG):
            sl = pl.ds(k * 16, 16)
            sb0[sl] = eb0[1, sl] + noff
        pltpu.async_copy(p1_hbm.at[sb0], rows0, gsem0)

        def half(c, i):
            eb, esem = ebs[i], esems[i]
            n_eb, n_esem = ebs[(i + 1) % 4], esems[(i + 1) % 4]
            sb, gsem, rows = sbs[i % 2], gsems[i % 2], rowss[i % 2]
            n_sb, n_gsem, n_rows = (sbs[(i + 1) % 2], gsems[(i + 1) % 2],
                                    rowss[(i + 1) % 2])
            P = i // 2
            dx, cn, ssem = dxs[P], cns[P], ssems[P]
            off = (i % 2) * C

            pltpu.make_async_copy(e_hbm.at[base], n_eb, n_esem).wait()
            for k in range(NG):
                sl = pl.ds(k * 16, 16)
                n_sb[sl] = n_eb[1, sl] + noff
            pltpu.async_copy(p1_hbm.at[n_sb], n_rows, n_gsem)

            if i % 2 == 0:
                pltpu.make_async_copy(cn, h1_sh.at[dx], ssem).wait()

            for g2 in range(NG):
                sl = pl.ds(g2 * 16, 16)
                dv = eb[0, sl]
                z = (plsc.load_gather(s0_t, [dv])
                     + plsc.load_gather(s1_t, [eb[1, sl]])
                     + plsc.load_gather(sr_t, [eb[2, sl]])
                     + plsc.load_gather(sr_t, [eb[3, sl]]))
                e = jnp.exp(-jnp.where(z > 0, z, ALPHA * z))
                e_b[sl] = e
                plsc.addupdate_scatter(rowsum_t, [dv], e)
            tvs = []
            for g2 in range(NG):
                sl = pl.ds(g2 * 16, 16)
                dx[pl.ds(off + g2 * 16, 16)] = eb[0, sl]
                tvs.append((eb[2, sl], eb[3, sl]))
            gn = jnp.where(c + 4 < nch, c + 4, 0)
            pltpu.async_copy(e_hbm.at[base + gn], eb, esem)

            pltpu.make_async_copy(p1_hbm.at[sb], rows, gsem).wait()
            for g2 in range(NG):
                t0vec, t1vec = tvs[g2]
                for j in range(16):
                    cc = g2 * 16 + j
                    t0c = t0vec[j]
                    t1c = t1vec[j]
                    ev = plsc.load_gather(
                        e_b, [jnp.full((16,), cc, jnp.int32)])
                    for b in range(NB):
                        sl = pl.ds(b * 16, 16)
                        cn[off + cc, sl] = (rows[cc, sl] + pr_t[t0c, sl]
                                            + pr_t[t1c, sl]) * ev
            if i % 2 == 1:
                pltpu.async_copy(cn, h1_sh.at[dx], ssem, add=True)

        def quad(k, carry):
            c0 = 4 * k
            for i in range(4):
                half(c0 + i, i)
            return carry

        lax.fori_loop(0, nch // 4, quad, 0)

        pltpu.make_async_copy(e_hbm.at[base], eb1, esem1).wait()
        pltpu.make_async_copy(e_hbm.at[base], eb2, esem2).wait()
        pltpu.make_async_copy(e_hbm.at[base], eb3, esem3).wait()
        pltpu.make_async_copy(p1_hbm.at[sb0], rows0, gsem0).wait()
        pltpu.make_async_copy(cn0, h1_sh.at[dx0], ssem0).wait()
        pltpu.make_async_copy(cn1, h1_sh.at[dx1], ssem1).wait()

        plsc.subcore_barrier()
        pltpu.sync_copy(h1_sh.at[pl.ds(sid * ZR, ZR)],
                        h1_out.at[cid, pl.ds(sid * ZR, ZR)])
        pltpu.sync_copy(rowsum_t, rs_out.at[cid * 16 + sid])

    mesh = plsc.VectorSubcoreMesh(
        core_axis_name="c", subcore_axis_name="s", num_cores=2, num_subcores=16)
    return pl.kernel(
        body,
        out_type=[
            jax.ShapeDtypeStruct((2, NPAD, F), jnp.float32),
            jax.ShapeDtypeStruct((32, NPAD), jnp.float32),
        ],
        mesh=mesh,
        compiler_params=pltpu.CompilerParams(
            needs_layout_passes=False, use_tc_tiling_on_sc=False),
        scratch_types=[
            pltpu.VMEM((NPAD,), jnp.float32),
            pltpu.VMEM((NPAD,), jnp.float32),
            pltpu.VMEM((RPAD,), jnp.float32),
            pltpu.VMEM((NPAD,), jnp.float32),
            pltpu.VMEM((RPAD, F), jnp.float32),
            pltpu.VMEM((4, C), jnp.int32),
            pltpu.VMEM((4, C), jnp.int32),
            pltpu.VMEM((4, C), jnp.int32),
            pltpu.VMEM((4, C), jnp.int32),
            pltpu.VMEM((C,), jnp.int32),
            pltpu.VMEM((C,), jnp.int32),
            pltpu.VMEM((2 * C,), jnp.int32),
            pltpu.VMEM((2 * C,), jnp.int32),
            pltpu.VMEM((C, F), jnp.float32),
            pltpu.VMEM((C, F), jnp.float32),
            pltpu.VMEM((2 * C, F), jnp.float32),
            pltpu.VMEM((2 * C, F), jnp.float32),
            pltpu.VMEM((C,), jnp.float32),
            pltpu.VMEM_SHARED((NPAD, F), jnp.float32),
            pltpu.SemaphoreType.DMA,
            pltpu.SemaphoreType.DMA,
            pltpu.SemaphoreType.DMA,
            pltpu.SemaphoreType.DMA,
            pltpu.SemaphoreType.DMA,
            pltpu.SemaphoreType.DMA,
            pltpu.SemaphoreType.DMA,
            pltpu.SemaphoreType.DMA,
        ],
    )


def _norm_rows(x):
    n = jnp.sqrt(jnp.sum(x * x, axis=1, keepdims=True))
    return x / jnp.maximum(n, 1e-12)


def _elu(x):
    return jnp.where(x > 0, x, jnp.exp(x) - 1.0)


def _node_dense(ent, w_ent, aa, s):
    RB = 1000

    def body(ent_ref, w_ref, aa_ref, s_ref, eup_ref, pall_ref, sall_ref):
        xn = _norm_rows(ent_ref[...])
        eup_ref[...] = jnp.dot(xn, w_ref[...], preferred_element_type=jnp.float32)
        p = jnp.dot(xn, aa_ref[...], preferred_element_type=jnp.float32)
        pall_ref[...] = p
        sall_ref[...] = jnp.dot(p, s_ref[...], preferred_element_type=jnp.float32)

    return pl.pallas_call(
        body,
        grid=(N_NODES // RB,),
        in_specs=[
            pl.BlockSpec((RB, ENT_IN), lambda i: (i, 0)),
            pl.BlockSpec((ENT_IN, OUT_DIM), lambda i: (0, 0)),
            pl.BlockSpec((ENT_IN, 256), lambda i: (0, 0)),
            pl.BlockSpec((256, 8), lambda i: (0, 0)),
        ],
        out_specs=[
            pl.BlockSpec((RB, OUT_DIM), lambda i: (i, 0)),
            pl.BlockSpec((RB, 256), lambda i: (i, 0)),
            pl.BlockSpec((RB, 8), lambda i: (i, 0)),
        ],
        out_shape=[
            jax.ShapeDtypeStruct((N_NODES, OUT_DIM), jnp.float32),
            jax.ShapeDtypeStruct((N_NODES, 256), jnp.float32),
            jax.ShapeDtypeStruct((N_NODES, 8), jnp.float32),
        ],
    )(ent, w_ent, aa, s)


def _rel_dense(rel, w_ent, w_gat, a2cat, a2b, ssr):
    def body(rel_ref, went_ref, wgat_ref, a2c_ref, a2b_ref, ssr_ref,
             outrel_ref, pr12_ref, prb_ref, sr_ref):
        xn = _norm_rows(rel_ref[...])
        rup = jnp.dot(xn, went_ref[...], preferred_element_type=jnp.float32)
        orel1 = jnp.dot(xn, wgat_ref[...], preferred_element_type=jnp.float32)
        outrel_ref[...] = rup + orel1
        pr12 = jnp.dot(xn, a2c_ref[...], preferred_element_type=jnp.float32)
        pr12_ref[...] = pr12
        prb = jnp.dot(orel1, a2b_ref[...], preferred_element_type=jnp.float32)
        prb_ref[...] = prb
        q = jnp.concatenate([pr12, prb], axis=1)
        sr_ref[...] = jnp.dot(q, ssr_ref[...], preferred_element_type=jnp.float32)

    return pl.pallas_call(
        body,
        out_shape=[
            jax.ShapeDtypeStruct((N_REL, OUT_DIM), jnp.float32),
            jax.ShapeDtypeStruct((N_REL, 128), jnp.float32),
            jax.ShapeDtypeStruct((N_REL, OUT_DIM), jnp.float32),
            jax.ShapeDtypeStruct((N_REL, 8), jnp.float32),
        ],
    )(rel, w_ent, w_gat, a2cat, a2b, ssr)


def _mid_dense(h1, rsp, pall, aab, sb):
    RB = 1000

    def body(h1_ref, rsp_ref, pall_ref, aab_ref, sb_ref, pb_ref, sball_ref):
        rs = jnp.sum(rsp_ref[...], axis=2)
        p = pall_ref[...]
        xs = []
        for h in (0, 1):
            p0 = p[:, 128 * h:128 * h + NHID]
            h1h = h1_ref[h]
            rsh = rs[h][:, None]
            guard = jnp.where(rsh == 0, 1.0, rsh)
            hp = jnp.where(rsh == 0, 0.0, p0 + h1h / guard)
            xs.append(_elu(hp))
        x = jnp.concatenate(xs, axis=1)
        pb = jnp.dot(x, aab_ref[...], preferred_element_type=jnp.float32)
        pb_ref[...] = pb
        sball_ref[...] = jnp.dot(pb, sb_ref[...], preferred_element_type=jnp.float32)

    return pl.pallas_call(
        body,
        grid=(N_NODES // RB,),
        in_specs=[
            pl.BlockSpec((2, RB, NHID), lambda i: (0, i, 0)),
            pl.BlockSpec((2, RB, 16), lambda i: (0, i, 0)),
            pl.BlockSpec((RB, 256), lambda i: (i, 0)),
            pl.BlockSpec((OUT_DIM, 256), lambda i: (0, 0)),
            pl.BlockSpec((256, 8), lambda i: (0, 0)),
        ],
        out_specs=[
            pl.BlockSpec((RB, 256), lambda i: (i, 0)),
            pl.BlockSpec((RB, 8), lambda i: (i, 0)),
        ],
        out_shape=[
            jax.ShapeDtypeStruct((N_NODES, 256), jnp.float32),
            jax.ShapeDtypeStruct((N_NODES, 8), jnp.float32),
        ],
    )(h1, rsp, pall, aab, sb)


def _final_dense(h1b, rsp2, pball, eup, bi2):
    RB = 1000

    def body(h1b_ref, rsp2_ref, pb_ref, eup_ref, bi_ref, out_ref):
        i = pl.program_id(0)
        h1 = jnp.concatenate([h1b_ref[0], h1b_ref[1]], axis=1)
        rs = jnp.sum(rsp2_ref[...], axis=1)[:, None]
        p0b = pb_ref[:, :OUT_DIM]
        guard = jnp.where(rs == 0, 1.0, rs)
        hp = jnp.where(rs == 0, 0.0, p0b + h1 / guard)
        x2 = _elu(hp)
        nid = lax.broadcasted_iota(jnp.int32, (1, RB), 1) + i * RB
        acc = jnp.zeros((1, RB), jnp.bool_)
        bi = bi_ref[...]
        for k in range(32):
            eq = bi[k, :][:, None] == nid
            acc = acc | jnp.any(eq, axis=0, keepdims=True)
        mask = acc.astype(jnp.float32).T
        oe = eup_ref[...] + mask * x2
        out_ref[...] = _norm_rows(oe)

    return pl.pallas_call(
        body,
        grid=(N_NODES // RB,),
        in_specs=[
            pl.BlockSpec((2, RB, NHID), lambda i: (0, i, 0)),
            pl.BlockSpec((RB, 16), lambda i: (i, 0)),
            pl.BlockSpec((RB, 256), lambda i: (i, 0)),
            pl.BlockSpec((RB, OUT_DIM), lambda i: (i, 0)),
            pl.BlockSpec((32, 128), lambda i: (0, 0)),
        ],
        out_specs=pl.BlockSpec((RB, OUT_DIM), lambda i: (i, 0)),
        out_shape=jax.ShapeDtypeStruct((N_NODES, OUT_DIM), jnp.float32),
    )(h1b, rsp2, pball, eup, bi2)


def _pad_nodes(x):
    pad = [(0, 0)] * x.ndim
    pad[-2] = (0, NPAD - N_NODES)
    return jnp.pad(x, pad)


def kernel(edge_list, edge_type, batch_inputs, train_indices_nhop,
           entity_embeddings, relation_embeddings, W_entities, W_gat,
           a_head0, a2_head0, a_head1, a2_head1, a_out, a2_out):
    f32 = jnp.float32

    def split_a(a, fin):
        return a[:, :fin], a[:, fin:2 * fin], a[:, 2 * fin:]

    A0h0, A1h0, A2h0 = split_a(a_head0, ENT_IN)
    A0h1, A1h1, A2h1 = split_a(a_head1, ENT_IN)
    A0b, A1b, A2b = split_a(a_out, OUT_DIM)

    AA = jnp.concatenate([A0h0.T, A1h0.T, A0h1.T, A1h1.T], axis=1)
    S = jnp.zeros((256, 8), f32)
    S = S.at[0:64, 0].set(a2_head0[0]).at[64:128, 1].set(a2_head0[0])
    S = S.at[128:192, 2].set(a2_head1[0]).at[192:256, 3].set(a2_head1[0])

    A2cat = jnp.concatenate([A2h0.T, A2h1.T], axis=1)
    Ssr = jnp.zeros((256, 8), f32)
    Ssr = Ssr.at[0:64, 0].set(a2_head0[0]).at[64:128, 1].set(a2_head1[0])
    Ssr = Ssr.at[128:256, 2].set(a2_out[0])

    AAb = jnp.concatenate([A0b.T, A1b.T], axis=1)
    Sb = jnp.zeros((256, 8), f32)
    Sb = Sb.at[0:128, 0].set(a2_out[0]).at[128:256, 1].set(a2_out[0])

    tin = train_indices_nhop
    d = jnp.concatenate([edge_list[0], tin[:, 3]])
    s = jnp.concatenate([edge_list[1], tin[:, 0]])
    t0 = jnp.concatenate([edge_type, tin[:, 1]])
    t1 = jnp.concatenate([jnp.full((N_EDGES,), PAD_TYPE, jnp.int32), tin[:, 2]])

    def pad_e(x, val):
        return jnp.pad(x, (0, E_PAD - E_TOT), constant_values=val)

    d = pad_e(d, PAD_NODE)
    s = pad_e(s, PAD_NODE)
    t0 = pad_e(t0, PAD_TYPE)
    t1 = pad_e(t1, PAD_TYPE)

    epk = jnp.stack([x.reshape(NSL1, NCH1, C) for x in (d, s, t0, t1)],
                    axis=2).reshape(NSL1 * NCH1, 4, C)

    eup, pall, sall = _node_dense(entity_embeddings, W_entities, AA, S)
    out_relation, pr12, prb, srs = _rel_dense(
        relation_embeddings, W_entities, W_gat, A2cat, A2b.T, Ssr)

    p1pair = _pad_nodes(jnp.stack([pall[:, 64:128], pall[:, 192:256]]))
    p1flat = p1pair.reshape(2 * NPAD, NHID)
    prpair = jnp.stack([pr12[:, :64], pr12[:, 64:]])
    prflat = jnp.pad(prpair, ((0, 0), (0, RPAD - N_REL), (0, 0)))
    s0p = _pad_nodes(jnp.stack([sall[:, 0], sall[:, 2]]).T).T.reshape(-1)
    s1p = _pad_nodes(jnp.stack([sall[:, 1], sall[:, 3]]).T).T.reshape(-1)
    srp = jnp.pad(jnp.stack([srs[:, 0], srs[:, 1]]).T,
                  ((0, RPAD - N_REL), (0, 0))).T.reshape(-1)

    h1, rsp = _make_edge_kernel(NHID, NSL1, NCH1)(
        epk, s0p, s1p, srp, p1flat, prflat)

    rspT = rsp.reshape(2, 16, NPAD).transpose(0, 2, 1)
    pball, sball = _mid_dense(h1, rspT, pall, AAb, Sb)

    p1b_h = jnp.stack([pball[:, 128:192], pball[:, 192:256]])
    p1b = _pad_nodes(p1b_h).reshape(2 * NPAD, NHID)
    prb_h = jnp.stack([prb[:, :64], prb[:, 64:]])
    prb_p = jnp.pad(prb_h, ((0, 0), (0, RPAD - N_REL), (0, 0)))
    s0b = jnp.tile(_pad_nodes(sball[:, 0:1]).T[0], 2)
    s1b = jnp.tile(_pad_nodes(sball[:, 1:2]).T[0], 2)
    srb = jnp.tile(jnp.pad(srs[:, 2], (0, RPAD - N_REL)), 2)

    h1b, rsp2 = _make_edge_kernel(NHID, NSL1, NCH1)(
        epk, s0b, s1b, srb, p1b, prb_p)

    bi2 = batch_inputs[:, 2].reshape(32, 128)
    rsp2T = rsp2[:16].transpose(1, 0)
    out_entity = _final_dense(h1b, rsp2T, pball, eup, bi2)

    return out_entity, out_relation


# speedup vs baseline: 1.2119x; 1.2119x over previous
import functools

import jax
import jax.numpy as jnp
from jax import lax
from jax.experimental import pallas as pl
from jax.experimental.pallas import tpu as pltpu
from jax.experimental.pallas import tpu_sc as plsc

N_NODES = 10000
N_REL = 500
ENT_IN = 128
NHID = 64
NHEADS = 2
OUT_DIM = 128
ALPHA = 0.2
N_EDGES = 160000
N_NHOP = 20000
E_TOT = N_EDGES + N_NHOP

NPAD = 10112
RPAD = 512
PAD_NODE = N_NODES
PAD_TYPE = N_REL

C = 128
E_PAD = 180224
NSL1, NCH1 = 16, 88
ZR = NPAD // 16
BE = 1024
NBLK = E_PAD // BE

@functools.cache
def _make_edge_kernel(F, nsl, nch):
    NB = F // 16
    NG = C // 16

    def body(e_hbm, s0_hbm, s1_hbm, sr_hbm, p1_hbm, rel_hbm,
             h1_out, rs_out,
             s0_t, s1_t, sr_t, rowsum_t,
             ebA, ebB, sbA, sbB, dxA, dxB, rowsA, rowsB, relA, relB,
             cnA, cnB, e_bA, e_bB,
             h1_sh, esemA, esemB, gsemA, gsemB, ssemA, ssemB):
        cid = lax.axis_index("c")
        sid = lax.axis_index("s")
        base = sid * nch
        noff = cid * NPAD

        pltpu.sync_copy(s0_hbm.at[pl.ds(cid * NPAD, NPAD)], s0_t)
        pltpu.sync_copy(s1_hbm.at[pl.ds(cid * NPAD, NPAD)], s1_t)
        pltpu.sync_copy(sr_hbm.at[pl.ds(cid * RPAD, RPAD)], sr_t)

        zv = jnp.zeros((16,), jnp.float32)

        def zero_rowsum(i, carry):
            rowsum_t[pl.ds(i * 16, 16)] = zv
            return carry

        lax.fori_loop(0, NPAD // 16, zero_rowsum, 0)

        def zero_rows(i, carry):
            r = i // NB
            b = i % NB
            rowsA[r, pl.ds(b * 16, 16)] = zv
            return carry

        lax.fori_loop(0, C * NB, zero_rows, 0)

        nfull = ZR // C
        for k in range(nfull):
            pltpu.sync_copy(rowsA, h1_sh.at[pl.ds(sid * ZR + k * C, C)])
        if ZR % C:
            pltpu.sync_copy(rowsA.at[pl.ds(0, ZR % C)],
                            h1_sh.at[pl.ds(sid * ZR + nfull * C, ZR % C)])
        plsc.subcore_barrier()

        padv = jnp.full((16,), PAD_NODE, jnp.int32)
        for k in range(NG):
            dxA[pl.ds(k * 16, 16)] = padv
            dxB[pl.ds(k * 16, 16)] = padv
        pltpu.async_copy(cnA, h1_sh.at[dxA], ssemA, add=True)
        pltpu.async_copy(cnB, h1_sh.at[dxB], ssemB, add=True)
        pltpu.async_copy(e_hbm.at[base], ebA, esemA)
        pltpu.async_copy(e_hbm.at[base + 1], ebB, esemB)
        pltpu.make_async_copy(e_hbm.at[base], ebA, esemA).wait()
        for k in range(NG):
            sl = pl.ds(k * 16, 16)
            sbA[sl] = ebA[1, sl] + noff
        pltpu.async_copy(p1_hbm.at[sbA], rowsA, gsemA)
        pltpu.async_copy(rel_hbm.at[cid, pl.ds(base * C, C)], relA, gsemA)

        def half(c, eb, sb, dx, rows, rel, cn, e_b, esem, gsem, ssem,
                 o_eb, o_sb, o_rows, o_rel, o_esem, o_gsem):
            pltpu.make_async_copy(e_hbm.at[base], o_eb, o_esem).wait()
            for k in range(NG):
                sl = pl.ds(k * 16, 16)
                o_sb[sl] = o_eb[1, sl] + noff
            pltpu.async_copy(p1_hbm.at[o_sb], o_rows, o_gsem)
            pltpu.async_copy(rel_hbm.at[cid, pl.ds((base + c + 1) * C, C)],
                             o_rel, o_gsem)

            pltpu.make_async_copy(cn, h1_sh.at[dx], ssem).wait()

            for g2 in range(NG):
                sl = pl.ds(g2 * 16, 16)
                dv = eb[0, sl]
                z = (plsc.load_gather(s0_t, [dv])
                     + plsc.load_gather(s1_t, [eb[1, sl]])
                     + plsc.load_gather(sr_t, [eb[2, sl]])
                     + plsc.load_gather(sr_t, [eb[3, sl]]))
                e = jnp.exp(-jnp.where(z > 0, z, ALPHA * z))
                e_b[sl] = e
                plsc.addupdate_scatter(rowsum_t, [dv], e)
            for g2 in range(NG):
                sl = pl.ds(g2 * 16, 16)
                dx[sl] = eb[0, sl]
            g2next = jnp.where(c + 2 < nch, c + 2, 0)
            pltpu.async_copy(e_hbm.at[base + g2next], eb, esem)

            pltpu.make_async_copy(p1_hbm.at[sb], rows, gsem).wait()
            pltpu.make_async_copy(rel_hbm.at[cid, pl.ds(base * C, C)],
                                  rel, gsem).wait()
            for g2 in range(NG):
                for j in range(16):
                    cc = g2 * 16 + j
                    ev = plsc.load_gather(
                        e_b, [jnp.full((16,), cc, jnp.int32)])
                    for b in range(NB):
                        sl = pl.ds(b * 16, 16)
                        cn[cc, sl] = (rows[cc, sl] + rel[cc, sl]) * ev
            pltpu.async_copy(cn, h1_sh.at[dx], ssem, add=True)

        def pair(k, carry):
            cA = 2 * k
            half(cA, ebA, sbA, dxA, rowsA, relA, cnA, e_bA,
                 esemA, gsemA, ssemA,
                 ebB, sbB, rowsB, relB, esemB, gsemB)
            half(cA + 1, ebB, sbB, dxB, rowsB, relB, cnB, e_bB,
                 esemB, gsemB, ssemB,
                 ebA, sbA, rowsA, relA, esemA, gsemA)
            return carry

        lax.fori_loop(0, nch // 2, pair, 0)

        pltpu.make_async_copy(e_hbm.at[base], ebB, esemB).wait()
        pltpu.make_async_copy(p1_hbm.at[sbA], rowsA, gsemA).wait()
        pltpu.make_async_copy(rel_hbm.at[cid, pl.ds(base * C, C)],
                              relA, gsemA).wait()
        pltpu.make_async_copy(cnA, h1_sh.at[dxA], ssemA).wait()
        pltpu.make_async_copy(cnB, h1_sh.at[dxB], ssemB).wait()

        plsc.subcore_barrier()
        pltpu.sync_copy(h1_sh.at[pl.ds(sid * ZR, ZR)],
                        h1_out.at[cid, pl.ds(sid * ZR, ZR)])
        pltpu.sync_copy(rowsum_t, rs_out.at[cid * 16 + sid])

    mesh = plsc.VectorSubcoreMesh(
        core_axis_name="c", subcore_axis_name="s", num_cores=2, num_subcores=16)
    return pl.kernel(
        body,
        out_type=[
            jax.ShapeDtypeStruct((2, NPAD, F), jnp.float32),
            jax.ShapeDtypeStruct((32, NPAD), jnp.float32),
        ],
        mesh=mesh,
        compiler_params=pltpu.CompilerParams(
            needs_layout_passes=False, use_tc_tiling_on_sc=False),
        scratch_types=[
            pltpu.VMEM((NPAD,), jnp.float32),
            pltpu.VMEM((NPAD,), jnp.float32),
            pltpu.VMEM((RPAD,), jnp.float32),
            pltpu.VMEM((NPAD,), jnp.float32),
            pltpu.VMEM((4, C), jnp.int32),
            pltpu.VMEM((4, C), jnp.int32),
            pltpu.VMEM((C,), jnp.int32),
            pltpu.VMEM((C,), jnp.int32),
            pltpu.VMEM((C,), jnp.int32),
            pltpu.VMEM((C,), jnp.int32),
            pltpu.VMEM((C, F), jnp.float32),
            pltpu.VMEM((C, F), jnp.float32),
            pltpu.VMEM((C, F), jnp.float32),
            pltpu.VMEM((C, F), jnp.float32),
            pltpu.VMEM((C, F), jnp.float32),
            pltpu.VMEM((C, F), jnp.float32),
            pltpu.VMEM((C,), jnp.float32),
            pltpu.VMEM((C,), jnp.float32),
            pltpu.VMEM_SHARED((NPAD, F), jnp.float32),
            pltpu.SemaphoreType.DMA,
            pltpu.SemaphoreType.DMA,
            pltpu.SemaphoreType.DMA,
            pltpu.SemaphoreType.DMA,
            pltpu.SemaphoreType.DMA,
            pltpu.SemaphoreType.DMA,
        ],
    )


def _rel_rows(t0r, t1r, prpad):
    def body(t0_ref, t1_ref, pr_ref, rel_ref):
        t0v = t0_ref[0, 0, :]
        t1v = t1_ref[0, 0, :]
        io = lax.broadcasted_iota(jnp.int32, (BE, RPAD), 1)
        w = ((t0v[:, None] == io).astype(jnp.float32)
             + (t1v[:, None] == io).astype(jnp.float32))
        rel_ref[0] = jnp.dot(w, pr_ref[0],
                             preferred_element_type=jnp.float32)

    return pl.pallas_call(
        body,
        grid=(2, NBLK),
        in_specs=[
            pl.BlockSpec((1, 1, BE), lambda i, j: (j, 0, 0)),
            pl.BlockSpec((1, 1, BE), lambda i, j: (j, 0, 0)),
            pl.BlockSpec((1, RPAD, NHID), lambda i, j: (i, 0, 0)),
        ],
        out_specs=pl.BlockSpec((1, BE, NHID), lambda i, j: (i, j, 0)),
        out_shape=jax.ShapeDtypeStruct((2, E_PAD, NHID), jnp.float32),
    )(t0r, t1r, prpad)


def _norm_rows(x):
    n = jnp.sqrt(jnp.sum(x * x, axis=1, keepdims=True))
    return x / jnp.maximum(n, 1e-12)


def _elu(x):
    return jnp.where(x > 0, x, jnp.exp(x) - 1.0)


def _node_dense(ent, w_ent, aa, s):
    RB = 1000

    def body(ent_ref, w_ref, aa_ref, s_ref, eup_ref, pall_ref, sall_ref):
        xn = _norm_rows(ent_ref[...])
        eup_ref[...] = jnp.dot(xn, w_ref[...], preferred_element_type=jnp.float32)
        p = jnp.dot(xn, aa_ref[...], preferred_element_type=jnp.float32)
        pall_ref[...] = p
        sall_ref[...] = jnp.dot(p, s_ref[...], preferred_element_type=jnp.float32)

    return pl.pallas_call(
        body,
        grid=(N_NODES // RB,),
        in_specs=[
            pl.BlockSpec((RB, ENT_IN), lambda i: (i, 0)),
            pl.BlockSpec((ENT_IN, OUT_DIM), lambda i: (0, 0)),
            pl.BlockSpec((ENT_IN, 256), lambda i: (0, 0)),
            pl.BlockSpec((256, 8), lambda i: (0, 0)),
        ],
        out_specs=[
            pl.BlockSpec((RB, OUT_DIM), lambda i: (i, 0)),
            pl.BlockSpec((RB, 256), lambda i: (i, 0)),
            pl.BlockSpec((RB, 8), lambda i: (i, 0)),
        ],
        out_shape=[
            jax.ShapeDtypeStruct((N_NODES, OUT_DIM), jnp.float32),
            jax.ShapeDtypeStruct((N_NODES, 256), jnp.float32),
            jax.ShapeDtypeStruct((N_NODES, 8), jnp.float32),
        ],
    )(ent, w_ent, aa, s)


def _rel_dense(rel, w_ent, w_gat, a2cat, a2b, ssr):
    def body(rel_ref, went_ref, wgat_ref, a2c_ref, a2b_ref, ssr_ref,
             outrel_ref, pr12_ref, prb_ref, sr_ref):
        xn = _norm_rows(rel_ref[...])
        rup = jnp.dot(xn, went_ref[...], preferred_element_type=jnp.float32)
        orel1 = jnp.dot(xn, wgat_ref[...], preferred_element_type=jnp.float32)
        outrel_ref[...] = rup + orel1
        pr12 = jnp.dot(xn, a2c_ref[...], preferred_element_type=jnp.float32)
        pr12_ref[...] = pr12
        prb = jnp.dot(orel1, a2b_ref[...], preferred_element_type=jnp.float32)
        prb_ref[...] = prb
        q = jnp.concatenate([pr12, prb], axis=1)
        sr_ref[...] = jnp.dot(q, ssr_ref[...], preferred_element_type=jnp.float32)

    return pl.pallas_call(
        body,
        out_shape=[
            jax.ShapeDtypeStruct((N_REL, OUT_DIM), jnp.float32),
            jax.ShapeDtypeStruct((N_REL, 128), jnp.float32),
            jax.ShapeDtypeStruct((N_REL, OUT_DIM), jnp.float32),
            jax.ShapeDtypeStruct((N_REL, 8), jnp.float32),
        ],
    )(rel, w_ent, w_gat, a2cat, a2b, ssr)


def _mid_dense(h1, rsp, pall, aab, sb):
    RB = 1000

    def body(h1_ref, rsp_ref, pall_ref, aab_ref, sb_ref, pb_ref, sball_ref):
        rs = jnp.sum(rsp_ref[...], axis=2)
        p = pall_ref[...]
        xs = []
        for h in (0, 1):
            p0 = p[:, 128 * h:128 * h + NHID]
            h1h = h1_ref[h]
            rsh = rs[h][:, None]
            guard = jnp.where(rsh == 0, 1.0, rsh)
            hp = jnp.where(rsh == 0, 0.0, p0 + h1h / guard)
            xs.append(_elu(hp))
        x = jnp.concatenate(xs, axis=1)
        pb = jnp.dot(x, aab_ref[...], preferred_element_type=jnp.float32)
        pb_ref[...] = pb
        sball_ref[...] = jnp.dot(pb, sb_ref[...], preferred_element_type=jnp.float32)

    return pl.pallas_call(
        body,
        grid=(N_NODES // RB,),
        in_specs=[
            pl.BlockSpec((2, RB, NHID), lambda i: (0, i, 0)),
            pl.BlockSpec((2, RB, 16), lambda i: (0, i, 0)),
            pl.BlockSpec((RB, 256), lambda i: (i, 0)),
            pl.BlockSpec((OUT_DIM, 256), lambda i: (0, 0)),
            pl.BlockSpec((256, 8), lambda i: (0, 0)),
        ],
        out_specs=[
            pl.BlockSpec((RB, 256), lambda i: (i, 0)),
            pl.BlockSpec((RB, 8), lambda i: (i, 0)),
        ],
        out_shape=[
            jax.ShapeDtypeStruct((N_NODES, 256), jnp.float32),
            jax.ShapeDtypeStruct((N_NODES, 8), jnp.float32),
        ],
    )(h1, rsp, pall, aab, sb)


def _final_dense(h1b, rsp2, pball, eup, bi2):
    RB = 1000

    def body(h1b_ref, rsp2_ref, pb_ref, eup_ref, bi_ref, out_ref):
        i = pl.program_id(0)
        h1 = jnp.concatenate([h1b_ref[0], h1b_ref[1]], axis=1)
        rs = jnp.sum(rsp2_ref[...], axis=1)[:, None]
        p0b = pb_ref[:, :OUT_DIM]
        guard = jnp.where(rs == 0, 1.0, rs)
        hp = jnp.where(rs == 0, 0.0, p0b + h1 / guard)
        x2 = _elu(hp)
        nid = lax.broadcasted_iota(jnp.int32, (1, RB), 1) + i * RB
        acc = jnp.zeros((1, RB), jnp.bool_)
        bi = bi_ref[...]
        for k in range(32):
            eq = bi[k, :][:, None] == nid
            acc = acc | jnp.any(eq, axis=0, keepdims=True)
        mask = acc.astype(jnp.float32).T
        oe = eup_ref[...] + mask * x2
        out_ref[...] = _norm_rows(oe)

    return pl.pallas_call(
        body,
        grid=(N_NODES // RB,),
        in_specs=[
            pl.BlockSpec((2, RB, NHID), lambda i: (0, i, 0)),
            pl.BlockSpec((RB, 16), lambda i: (i, 0)),
            pl.BlockSpec((RB, 256), lambda i: (i, 0)),
            pl.BlockSpec((RB, OUT_DIM), lambda i: (i, 0)),
            pl.BlockSpec((32, 128), lambda i: (0, 0)),
        ],
        out_specs=pl.BlockSpec((RB, OUT_DIM), lambda i: (i, 0)),
        out_shape=jax.ShapeDtypeStruct((N_NODES, OUT_DIM), jnp.float32),
    )(h1b, rsp2, pball, eup, bi2)


def _pad_nodes(x):
    pad = [(0, 0)] * x.ndim
    pad[-2] = (0, NPAD - N_NODES)
    return jnp.pad(x, pad)


def kernel(edge_list, edge_type, batch_inputs, train_indices_nhop,
           entity_embeddings, relation_embeddings, W_entities, W_gat,
           a_head0, a2_head0, a_head1, a2_head1, a_out, a2_out):
    f32 = jnp.float32

    def split_a(a, fin):
        return a[:, :fin], a[:, fin:2 * fin], a[:, 2 * fin:]

    A0h0, A1h0, A2h0 = split_a(a_head0, ENT_IN)
    A0h1, A1h1, A2h1 = split_a(a_head1, ENT_IN)
    A0b, A1b, A2b = split_a(a_out, OUT_DIM)

    AA = jnp.concatenate([A0h0.T, A1h0.T, A0h1.T, A1h1.T], axis=1)
    S = jnp.zeros((256, 8), f32)
    S = S.at[0:64, 0].set(a2_head0[0]).at[64:128, 1].set(a2_head0[0])
    S = S.at[128:192, 2].set(a2_head1[0]).at[192:256, 3].set(a2_head1[0])

    A2cat = jnp.concatenate([A2h0.T, A2h1.T], axis=1)
    Ssr = jnp.zeros((256, 8), f32)
    Ssr = Ssr.at[0:64, 0].set(a2_head0[0]).at[64:128, 1].set(a2_head1[0])
    Ssr = Ssr.at[128:256, 2].set(a2_out[0])

    AAb = jnp.concatenate([A0b.T, A1b.T], axis=1)
    Sb = jnp.zeros((256, 8), f32)
    Sb = Sb.at[0:128, 0].set(a2_out[0]).at[128:256, 1].set(a2_out[0])

    tin = train_indices_nhop
    d = jnp.concatenate([edge_list[0], tin[:, 3]])
    s = jnp.concatenate([edge_list[1], tin[:, 0]])
    t0 = jnp.concatenate([edge_type, tin[:, 1]])
    t1 = jnp.concatenate([jnp.full((N_EDGES,), PAD_TYPE, jnp.int32), tin[:, 2]])

    def pad_e(x, val):
        return jnp.pad(x, (0, E_PAD - E_TOT), constant_values=val)

    d = pad_e(d, PAD_NODE)
    s = pad_e(s, PAD_NODE)
    t0 = pad_e(t0, PAD_TYPE)
    t1 = pad_e(t1, PAD_TYPE)

    epk = jnp.stack([x.reshape(NSL1, NCH1, C) for x in (d, s, t0, t1)],
                    axis=2).reshape(NSL1 * NCH1, 4, C)
    t0r = t0.reshape(NBLK, 1, BE)
    t1r = t1.reshape(NBLK, 1, BE)

    eup, pall, sall = _node_dense(entity_embeddings, W_entities, AA, S)
    out_relation, pr12, prb, srs = _rel_dense(
        relation_embeddings, W_entities, W_gat, A2cat, A2b.T, Ssr)

    p1pair = _pad_nodes(jnp.stack([pall[:, 64:128], pall[:, 192:256]]))
    p1flat = p1pair.reshape(2 * NPAD, NHID)
    prpair = jnp.stack([pr12[:, :64], pr12[:, 64:]])
    prflat = jnp.pad(prpair, ((0, 0), (0, RPAD - N_REL), (0, 0)))
    s0p = _pad_nodes(jnp.stack([sall[:, 0], sall[:, 2]]).T).T.reshape(-1)
    s1p = _pad_nodes(jnp.stack([sall[:, 1], sall[:, 3]]).T).T.reshape(-1)
    srp = jnp.pad(jnp.stack([srs[:, 0], srs[:, 1]]).T,
                  ((0, RPAD - N_REL), (0, 0))).T.reshape(-1)

    rel1 = _rel_rows(t0r, t1r, prflat)
    h1, rsp = _make_edge_kernel(NHID, NSL1, NCH1)(
        epk, s0p, s1p, srp, p1flat, rel1)

    rspT = rsp.reshape(2, 16, NPAD).transpose(0, 2, 1)
    pball, sball = _mid_dense(h1, rspT, pall, AAb, Sb)

    p1b_h = jnp.stack([pball[:, 128:192], pball[:, 192:256]])
    p1b = _pad_nodes(p1b_h).reshape(2 * NPAD, NHID)
    prb_h = jnp.stack([prb[:, :64], prb[:, 64:]])
    prb_p = jnp.pad(prb_h, ((0, 0), (0, RPAD - N_REL), (0, 0)))
    s0b = jnp.tile(_pad_nodes(sball[:, 0:1]).T[0], 2)
    s1b = jnp.tile(_pad_nodes(sball[:, 1:2]).T[0], 2)
    srb = jnp.tile(jnp.pad(srs[:, 2], (0, RPAD - N_REL)), 2)

    rel2 = _rel_rows(t0r, t1r, prb_p)
    h1b, rsp2 = _make_edge_kernel(NHID, NSL1, NCH1)(
        epk, s0b, s1b, srb, p1b, rel2)

    bi2 = batch_inputs[:, 2].reshape(32, 128)
    rsp2T = rsp2[:16].transpose(1, 0)
    out_entity = _final_dense(h1b, rsp2T, pball, eup, bi2)

    return out_entity, out_relation
